# baseline (speedup 1.0000x reference)
import jax
import jax.numpy as jnp
from jax.experimental import pallas as pl
from jax.experimental.pallas import tpu as pltpu

_DEPTH = 22
_ATOMS = 5
_ROW = _ATOMS * _DEPTH


def _body(x_ref, o_ref):
    x = x_ref[...]
    logits = (
        x[:, 0:_DEPTH]
        + x[:, _DEPTH : 2 * _DEPTH]
        + x[:, 2 * _DEPTH : 3 * _DEPTH]
        + x[:, 3 * _DEPTH : 4 * _DEPTH]
        + x[:, 4 * _DEPTH : 5 * _DEPTH]
    )
    m = jnp.max(logits, axis=1, keepdims=True)
    iota = jax.lax.broadcasted_iota(jnp.int32, logits.shape, 1)
    idx = jnp.min(jnp.where(logits == m, iota, _DEPTH), axis=1, keepdims=True)
    onehot = (iota == idx).astype(jnp.float32)
    o_ref[...] = jnp.concatenate([onehot] * _ATOMS, axis=1)


def kernel(inputs):
    b, l, a, d = inputs.shape
    rows = b * l
    x2 = inputs.reshape(rows, a * d)
    blk = 2048
    out = pl.pallas_call(
        _body,
        grid=(rows // blk,),
        in_specs=[pl.BlockSpec((blk, a * d), lambda i: (i, 0))],
        out_specs=pl.BlockSpec((blk, a * d), lambda i: (i, 0)),
        out_shape=jax.ShapeDtypeStruct((rows, a * d), jnp.float32),
    )(x2)
    return out.reshape(b, l, a, d)


# speedup vs baseline: 11.6891x; 11.6891x over previous
import jax
import jax.numpy as jnp
from jax.experimental import pallas as pl
from jax.experimental.pallas import tpu as pltpu

_DEPTH = 22
_ATOMS = 5


def _body(x_ref, o_ref):
    best = (
        x_ref[0, 0] + x_ref[1, 0] + x_ref[2, 0] + x_ref[3, 0] + x_ref[4, 0]
    )
    idx = jnp.zeros(best.shape, dtype=jnp.int32)
    for d in range(1, _DEPTH):
        sd = (
            x_ref[0, d] + x_ref[1, d] + x_ref[2, d] + x_ref[3, d] + x_ref[4, d]
        )
        gt = sd > best
        best = jnp.where(gt, sd, best)
        idx = jnp.where(gt, d, idx)
    for d in range(_DEPTH):
        oh = (idx == d).astype(jnp.float32)
        for a in range(_ATOMS):
            o_ref[a, d] = oh


def kernel(inputs):
    b, l, a, d = inputs.shape
    x_t = jnp.transpose(inputs, (2, 3, 0, 1))
    c = 256
    out_t = pl.pallas_call(
        _body,
        grid=(l // c,),
        in_specs=[pl.BlockSpec((a, d, b, c), lambda i: (0, 0, 0, i))],
        out_specs=pl.BlockSpec((a, d, b, c), lambda i: (0, 0, 0, i)),
        out_shape=jax.ShapeDtypeStruct((a, d, b, l), jnp.float32),
    )(x_t)
    return jnp.transpose(out_t, (2, 3, 0, 1))
